# baseline (speedup 1.0000x reference)
import jax
import jax.numpy as jnp
from jax.experimental import pallas as pl
from jax.experimental.pallas import tpu as pltpu

_C = 100000
_IGNORE = 0
_SMOOTH = 0.1
_CONF = 1.0 - _SMOOTH
_EPS = _SMOOTH / (_C - 1)
_BR = 64


def _body(tgt_ref, logit_ref, out_ref):
    i = pl.program_id(0)
    t = tgt_ref[...]
    col = jax.lax.broadcasted_iota(jnp.int32, (_BR, _C), 1)
    blk = logit_ref[...]
    s = jnp.sum(blk, axis=1, keepdims=True)
    g = jnp.sum(jnp.where(col == t, blk, 0.0), axis=1, keepdims=True)
    per_row = _EPS * s + (_CONF - _EPS) * g
    partial = jnp.sum(jnp.where(t != _IGNORE, per_row, 0.0))

    @pl.when(i == 0)
    def _init():
        out_ref[...] = jnp.zeros((1, 1), jnp.float32)

    out_ref[...] += jnp.full((1, 1), -partial, jnp.float32)


def kernel(logit, target):
    n = logit.shape[0]
    tgt = target.astype(jnp.int32).reshape(n, 1)
    out = pl.pallas_call(
        _body,
        grid=(n // _BR,),
        in_specs=[
            pl.BlockSpec((_BR, 1), lambda i: (i, 0)),
            pl.BlockSpec((_BR, _C), lambda i: (i, 0)),
        ],
        out_specs=pl.BlockSpec((1, 1), lambda i: (0, 0)),
        out_shape=jax.ShapeDtypeStruct((1, 1), jnp.float32),
        compiler_params=pltpu.CompilerParams(
            dimension_semantics=("arbitrary",),
        ),
    )(tgt, logit)
    return out[0, 0]
